# baseline (speedup 1.0000x reference)
import functools

import jax
import jax.numpy as jnp
from jax import lax
from jax.experimental import pallas as pl
from jax.experimental.pallas import tpu as pltpu
from jax.experimental.pallas import tpu_sc as plsc

_B = 16384


def _sc_gather(author, subreddit, comment, A_emb, S_emb, T_emb):
    info = plsc.get_sparse_core_info()
    nc, ns = info.num_cores, info.num_subcores
    nw = nc * ns
    bpw = _B // nw

    mesh = plsc.VectorSubcoreMesh(core_axis_name="c", subcore_axis_name="s")

    @functools.partial(
        pl.kernel,
        mesh=mesh,
        compiler_params=pltpu.CompilerParams(use_tc_tiling_on_sc=False),
        out_type=[
            jax.ShapeDtypeStruct((_B, 32), jnp.float32),
            jax.ShapeDtypeStruct((_B, 32), jnp.float32),
            jax.ShapeDtypeStruct((_B, 64), jnp.float32),
        ],
        scratch_types=[
            pltpu.VMEM((bpw,), jnp.int32),
            pltpu.VMEM((bpw,), jnp.int32),
            pltpu.VMEM((bpw,), jnp.int32),
            pltpu.VMEM((bpw, 32), jnp.float32),
            pltpu.VMEM((bpw, 32), jnp.float32),
            pltpu.VMEM((bpw, 64), jnp.float32),
            pltpu.SemaphoreType.DMA,
            pltpu.SemaphoreType.DMA,
            pltpu.SemaphoreType.DMA,
        ],
    )
    def gather_kernel(a_hbm, s_hbm, c_hbm, ae_hbm, se_hbm, te_hbm,
                      out_a, out_s, out_t,
                      ia, isv, ic, ra, rs, rt, sem_a, sem_s, sem_t):
        wid = lax.axis_index("s") * nc + lax.axis_index("c")
        base = wid * bpw
        pltpu.sync_copy(a_hbm.at[pl.ds(base, bpw)], ia)
        pltpu.sync_copy(s_hbm.at[pl.ds(base, bpw)], isv)
        pltpu.sync_copy(c_hbm.at[pl.ds(base, bpw)], ic)
        ca = pltpu.async_copy(ae_hbm.at[ia], ra, sem_a)
        cs = pltpu.async_copy(se_hbm.at[isv], rs, sem_s)
        ct = pltpu.async_copy(te_hbm.at[ic], rt, sem_t)
        ca.wait()
        pltpu.sync_copy(ra, out_a.at[pl.ds(base, bpw)])
        cs.wait()
        pltpu.sync_copy(rs, out_s.at[pl.ds(base, bpw)])
        ct.wait()
        pltpu.sync_copy(rt, out_t.at[pl.ds(base, bpw)])

    return gather_kernel(author, subreddit, comment, A_emb, S_emb, T_emb)


def _tc_body(ga, gs, gt, a_w, a_b, s_w, s_b, t_w1, t_w2, t_b1, t_b2,
             l1a, l1c, l1_b, l2_w, l2_b, out):
    f32 = jnp.float32
    ar = jnp.dot(ga[...], a_w[...], preferred_element_type=f32) + a_b[...]
    sr = jnp.dot(gs[...], s_w[...], preferred_element_type=f32) + s_b[...]
    cr1 = jnp.dot(gt[...], t_w1[...], preferred_element_type=f32) + t_b1[...]
    cr2 = jnp.dot(gt[...], t_w2[...], preferred_element_type=f32) + t_b2[...]
    m = (jnp.dot(ar * cr1, l1a[...], preferred_element_type=f32)
         + jnp.dot(sr * cr2, l1c[...], preferred_element_type=f32)
         + l1_b[...])
    o = jnp.dot(m, l2_w[...], preferred_element_type=f32) + l2_b[...]
    out[...] = o[:, 0]


def _tc_dense(ga, gs, gt, a_w, a_b, s_w, s_b, t_w1, t_w2, t_b1, t_b2,
              l1a, l1c, l1_b, l2_w, l2_b):
    blk = 2048
    grid = _B // blk

    def full(x):
        return pl.BlockSpec(x.shape, lambda i: (0,) * x.ndim)

    w_args = (a_w, a_b, s_w, s_b, t_w1, t_w2, t_b1, t_b2,
              l1a, l1c, l1_b, l2_w, l2_b)
    return pl.pallas_call(
        _tc_body,
        grid=(grid,),
        in_specs=[
            pl.BlockSpec((blk, 32), lambda i: (i, 0)),
            pl.BlockSpec((blk, 32), lambda i: (i, 0)),
            pl.BlockSpec((blk, 64), lambda i: (i, 0)),
            *[full(w) for w in w_args],
        ],
        out_specs=pl.BlockSpec((blk,), lambda i: (i,)),
        out_shape=jax.ShapeDtypeStruct((_B,), jnp.float32),
    )(ga, gs, gt, *w_args)


def kernel(author, subreddit, comment, A_emb, A_W, A_b, S_emb, S_W, S_b,
           T_emb, T_W, T_b, L1_W, L1_b, L2_W, L2_b):
    author = author.astype(jnp.int32)
    subreddit = subreddit.astype(jnp.int32)
    comment = comment.astype(jnp.int32)
    ga, gs, gt = _sc_gather(author, subreddit, comment, A_emb, S_emb, T_emb)
    t_w1, t_w2 = T_W[:, :50], T_W[:, 50:]
    t_b1, t_b2 = T_b[:50], T_b[50:]
    l1a, l1c = L1_W[:50, :], L1_W[50:, :]
    return _tc_dense(ga, gs, gt, A_W, A_b, S_W, S_b, t_w1, t_w2, t_b1, t_b2,
                     l1a, l1c, L1_b, L2_W, L2_b)
